# baseline (speedup 1.0000x reference)
import functools

import jax
import jax.numpy as jnp
from jax import lax
from jax.experimental import pallas as pl
from jax.experimental.pallas import tpu as pltpu
from jax.experimental.pallas import tpu_sc as plsc

B = 4096
L = 200
E = 64
AA_V = 22
POS_V = 210
NC = 2
NS = 16
NW = NC * NS
TPW = B * L // NW
RPW = B // NW
CHUNK = 128
NCHUNK = TPW // CHUNK
NBUF = 4


def _combine_body(aa_ref, pos_ref, out_ref):
    out_ref[...] = aa_ref[...][:, None, :] + pos_ref[...][None, :, :]


_combine = pl.pallas_call(
    _combine_body,
    out_shape=jax.ShapeDtypeStruct((AA_V, POS_V, E), jnp.float32),
)


def _embed_body(comb_hbm, seqs_hbm, lens_hbm, out_hbm,
                seq_v, len_v, idx_v, rows_v, gsem, osem):
    c_ax = lax.axis_index("c")
    s_ax = lax.axis_index("s")
    wid = s_ax * NC + c_ax
    tbase = wid * TPW

    pltpu.sync_copy(seqs_hbm.at[pl.ds(tbase, TPW)], seq_v)
    pltpu.sync_copy(lens_hbm.at[pl.ds(wid * RPW, RPW)], len_v)

    iota = lax.iota(jnp.int32, 16)

    def compute_idx(c):
        slot = lax.rem(c, NBUF)
        for g in range(CHUNK // 16):
            t0 = c * CHUNK + g * 16
            t = iota + t0
            s_vec = seq_v[pl.ds(t0, 16)]
            r_vec = t // L
            j_vec = t - r_vec * L
            len_vec = plsc.load_gather(len_v, [r_vec])
            jp1 = j_vec + 1
            p_vec = jnp.where(len_vec >= jp1, jp1, 0)
            idx_v[slot, pl.ds(g * 16, 16)] = s_vec * POS_V + p_vec

    def gather_desc(slot):
        return pltpu.make_async_copy(
            comb_hbm.at[idx_v.at[slot]], rows_v.at[slot], gsem)

    def scatter_desc(c, slot):
        return pltpu.make_async_copy(
            rows_v.at[slot],
            out_hbm.at[pl.ds(tbase + c * CHUNK, CHUNK)], osem)

    def chunk_body(c, carry):
        slot = lax.rem(c, NBUF)

        @pl.when(c >= NBUF)
        def _():
            scatter_desc(c, slot).wait()

        compute_idx(c)
        gather_desc(slot).start()

        @pl.when(c >= 1)
        def _():
            pslot = lax.rem(c - 1, NBUF)
            gather_desc(pslot).wait()
            scatter_desc(c - 1, pslot).start()
        return carry

    lax.fori_loop(0, NCHUNK, chunk_body, 0)

    last = NCHUNK - 1
    lslot = lax.rem(last, NBUF)
    gather_desc(lslot).wait()
    scatter_desc(last, lslot).start()
    for _ in range(NBUF):
        pltpu.make_async_copy(
            rows_v.at[0], out_hbm.at[pl.ds(tbase, CHUNK)], osem).wait()


@functools.partial(
    pl.kernel,
    out_type=jax.ShapeDtypeStruct((B * L, E), jnp.float32),
    mesh=plsc.VectorSubcoreMesh(core_axis_name="c", subcore_axis_name="s"),
    scratch_types=[
        pltpu.VMEM((TPW,), jnp.int32),
        pltpu.VMEM((RPW,), jnp.int32),
        pltpu.VMEM((NBUF, CHUNK), jnp.int32),
        pltpu.VMEM((NBUF, CHUNK, E), jnp.float32),
        pltpu.SemaphoreType.DMA,
        pltpu.SemaphoreType.DMA,
    ],
    compiler_params=pltpu.CompilerParams(
        needs_layout_passes=False, use_tc_tiling_on_sc=False),
)
def _embed(comb_hbm, seqs_hbm, lens_hbm, out_hbm,
           seq_v, len_v, idx_v, rows_v, gsem, osem):
    _embed_body(comb_hbm, seqs_hbm, lens_hbm, out_hbm,
                seq_v, len_v, idx_v, rows_v, gsem, osem)


def kernel(seqs, lens, aa_table, pos_table):
    comb = _combine(aa_table, pos_table).reshape(AA_V * POS_V, E)
    out = _embed(comb, seqs.reshape(B * L), lens)
    return out.reshape(B, L, E)


# speedup vs baseline: 3.4579x; 3.4579x over previous
import functools

import jax
import jax.numpy as jnp
from jax import lax
from jax.experimental import pallas as pl
from jax.experimental.pallas import tpu as pltpu
from jax.experimental.pallas import tpu_sc as plsc

B = 4096
L = 200
E = 64
AA_V = 22
POS_V = 210
NC = 2
NS = 16
NW = NC * NS
RPW = B // NW


def _embed_body(seqs_hbm, lens_hbm, aa_hbm, pos_hbm, out_hbm,
                aa_v, pos_v, seq_v, len_v, out_v, sem):
    c = lax.axis_index("c")
    s = lax.axis_index("s")
    wid = s * NC + c
    base = wid * RPW

    pltpu.sync_copy(aa_hbm, aa_v)
    pltpu.sync_copy(pos_hbm, pos_v)
    pltpu.sync_copy(seqs_hbm.at[pl.ds(base * L, RPW * L)], seq_v)
    pltpu.sync_copy(lens_hbm.at[pl.ds(base, RPW)], len_v)

    iota = lax.iota(jnp.int32, 16)
    cols = [iota + 16 * k for k in range(4)]

    def row_body(r, carry):
        row = base + r
        slot = lax.rem(r, 2)
        len_b = plsc.load_gather(len_v, [jnp.full((16,), r, jnp.int32)])
        t0 = r * L

        @pl.when(r >= 2)
        def _():
            pltpu.make_async_copy(out_v.at[slot], out_hbm.at[row], sem).wait()

        @plsc.parallel_loop(0, L, 1, unroll=4)
        def tok_body(j):
            s_b = plsc.load_gather(seq_v, [jnp.full((16,), t0 + j, jnp.int32)])
            jp1_b = jnp.full((16,), j + 1, jnp.int32)
            p_b = jnp.where(len_b >= jp1_b, jp1_b, 0)
            s64 = s_b << 6
            p64 = p_b << 6
            for k in range(4):
                a = plsc.load_gather(aa_v, [s64 + cols[k]])
                p = plsc.load_gather(pos_v, [p64 + cols[k]])
                out_v[slot, j, pl.ds(16 * k, 16)] = a + p
        pltpu.async_copy(out_v.at[slot], out_hbm.at[row], sem)
        return carry

    lax.fori_loop(0, RPW, row_body, 0)
    pltpu.make_async_copy(out_v.at[0], out_hbm.at[base], sem).wait()
    pltpu.make_async_copy(out_v.at[1], out_hbm.at[base], sem).wait()


@functools.partial(
    pl.kernel,
    out_type=jax.ShapeDtypeStruct((B, L, E), jnp.float32),
    mesh=plsc.VectorSubcoreMesh(core_axis_name="c", subcore_axis_name="s"),
    scratch_types=[
        pltpu.VMEM((AA_V * E,), jnp.float32),
        pltpu.VMEM((POS_V * E,), jnp.float32),
        pltpu.VMEM((RPW * L,), jnp.int32),
        pltpu.VMEM((RPW,), jnp.int32),
        pltpu.VMEM((2, L, E), jnp.float32),
        pltpu.SemaphoreType.DMA,
    ],
    compiler_params=pltpu.CompilerParams(
        needs_layout_passes=False, disable_bounds_checks=True),
)
def _embed(seqs_hbm, lens_hbm, aa_hbm, pos_hbm, out_hbm,
           aa_v, pos_v, seq_v, len_v, out_v, sem):
    _embed_body(seqs_hbm, lens_hbm, aa_hbm, pos_hbm, out_hbm,
                aa_v, pos_v, seq_v, len_v, out_v, sem)


def kernel(seqs, lens, aa_table, pos_table):
    return _embed(seqs.reshape(B * L), lens,
                  aa_table.reshape(AA_V * E), pos_table.reshape(POS_V * E))
